# Initial kernel scaffold; baseline (speedup 1.0000x reference)
import jax
import jax.numpy as jnp
from jax.experimental import pallas as pl


def kernel(x, Wg, bg, W1, b1, W2, b2, W3, b3):
    raise NotImplementedError("write your pallas kernel here")


# speedup vs baseline: 2.6850x; 2.6850x over previous
import functools

import jax
import jax.numpy as jnp
from jax import lax
from jax.experimental import pallas as pl
from jax.experimental.pallas import tpu as pltpu

S = 2048
D = 1024
H = 4096
NE = 8
TOPK = 2
BLK = 512
NBMAX = 16
RPAD = NBMAX * BLK
NT = 1024
KT = 1024


def _gelu(x):
    return 0.5 * x * (1.0 + lax.erf(x * 0.7071067811865476))


def _gating_body(x_ref, wg_ref, bg_ref, tv_ref, ti_ref):
    logits = lax.dot_general(
        x_ref[...], wg_ref[...], (((1,), (0,)), ((), ())),
        precision=lax.Precision.HIGHEST) + bg_ref[...]
    m = jnp.max(logits, axis=1, keepdims=True)
    e = jnp.exp(logits - m)
    p = e / jnp.sum(e, axis=1, keepdims=True)
    iota = lax.broadcasted_iota(jnp.int32, (S, NE), 1)
    m1 = jnp.max(p, axis=1, keepdims=True)
    i1 = jnp.min(jnp.where(p == m1, iota, NE), axis=1, keepdims=True)
    pm = jnp.where(iota == i1, -jnp.inf, p)
    m2 = jnp.max(pm, axis=1, keepdims=True)
    i2 = jnp.min(jnp.where(pm == m2, iota, NE), axis=1, keepdims=True)
    tv_ref[:, 0:1] = m1
    tv_ref[:, 1:2] = m2
    ti_ref[:, 0:1] = i1
    ti_ref[:, 1:2] = i2


def _gating(x2d, Wg, bg):
    return pl.pallas_call(
        _gating_body,
        out_shape=(
            jax.ShapeDtypeStruct((S, TOPK), jnp.float32),
            jax.ShapeDtypeStruct((S, TOPK), jnp.int32),
        ),
    )(x2d, Wg, bg.reshape(1, NE))


def _clamp(b, nv_ref):
    return jnp.minimum(b, nv_ref[0] - 1)


def _mlp1_body(be_ref, nv_ref, xs_ref, w1_ref, b1_ref, h1_ref):
    b = pl.program_id(0)

    @pl.when(b < nv_ref[0])
    def _():
        a = xs_ref[...].astype(jnp.bfloat16)
        w = w1_ref[0].astype(jnp.bfloat16)
        acc = jnp.dot(a, w, preferred_element_type=jnp.float32)
        h1_ref[...] = _gelu(acc + b1_ref[0])


def _mlp2_body(be_ref, nv_ref, h1_ref, w2_ref, b2_ref, h2_ref):
    b = pl.program_id(0)
    k = pl.program_id(2)
    nk = pl.num_programs(2)

    @pl.when(b < nv_ref[0])
    def _():
        off = pl.multiple_of(k * KT, KT)
        a = h1_ref[:, pl.ds(off, KT)].astype(jnp.bfloat16)
        part = jnp.dot(a, w2_ref[0].astype(jnp.bfloat16),
                       preferred_element_type=jnp.float32)

        @pl.when(k == 0)
        def _():
            h2_ref[...] = part

        @pl.when(k > 0)
        def _():
            h2_ref[...] += part

        @pl.when(k == nk - 1)
        def _():
            h2_ref[...] = _gelu(h2_ref[...] + b2_ref[0])


def _mlp3_body(be_ref, nv_ref, h2_ref, w3_ref, b3_ref, y_ref):
    b = pl.program_id(0)
    k = pl.program_id(1)
    nk = pl.num_programs(1)

    @pl.when(b < nv_ref[0])
    def _():
        off = pl.multiple_of(k * KT, KT)
        a = h2_ref[:, pl.ds(off, KT)].astype(jnp.bfloat16)
        part = jnp.dot(a, w3_ref[0].astype(jnp.bfloat16),
                       preferred_element_type=jnp.float32)

        @pl.when(k == 0)
        def _():
            y_ref[...] = part

        @pl.when(k > 0)
        def _():
            y_ref[...] += part

        @pl.when(k == nk - 1)
        def _():
            y_ref[...] += b3_ref[0]


def _grouped_mlp(xs, W1, b1, W2, b2, W3, b3, block_expert, nvalid):
    scalars = (block_expert, nvalid)

    h1 = pl.pallas_call(
        _mlp1_body,
        grid_spec=pltpu.PrefetchScalarGridSpec(
            num_scalar_prefetch=2,
            grid=(NBMAX, H // NT),
            in_specs=[
                pl.BlockSpec((BLK, D), lambda b, n, be, nv: (_clamp(b, nv), 0)),
                pl.BlockSpec((1, D, NT),
                             lambda b, n, be, nv: (be[_clamp(b, nv)], 0, n)),
                pl.BlockSpec((1, 1, NT),
                             lambda b, n, be, nv: (be[_clamp(b, nv)], 0, n)),
            ],
            out_specs=pl.BlockSpec((BLK, NT), lambda b, n, be, nv: (b, n)),
        ),
        out_shape=jax.ShapeDtypeStruct((RPAD, H), jnp.float32),
    )(*scalars, xs, W1, b1.reshape(NE, 1, H))

    h2 = pl.pallas_call(
        _mlp2_body,
        grid_spec=pltpu.PrefetchScalarGridSpec(
            num_scalar_prefetch=2,
            grid=(NBMAX, H // NT, H // KT),
            in_specs=[
                pl.BlockSpec((BLK, H),
                             lambda b, n, k, be, nv: (_clamp(b, nv), 0)),
                pl.BlockSpec((1, KT, NT),
                             lambda b, n, k, be, nv: (be[_clamp(b, nv)], k, n)),
                pl.BlockSpec((1, 1, NT),
                             lambda b, n, k, be, nv: (be[_clamp(b, nv)], 0, n)),
            ],
            out_specs=pl.BlockSpec((BLK, NT), lambda b, n, k, be, nv: (b, n)),
        ),
        out_shape=jax.ShapeDtypeStruct((RPAD, H), jnp.float32),
    )(*scalars, h1, W2, b2.reshape(NE, 1, H))

    y = pl.pallas_call(
        _mlp3_body,
        grid_spec=pltpu.PrefetchScalarGridSpec(
            num_scalar_prefetch=2,
            grid=(NBMAX, H // KT),
            in_specs=[
                pl.BlockSpec((BLK, H), lambda b, k, be, nv: (_clamp(b, nv), 0)),
                pl.BlockSpec((1, KT, D),
                             lambda b, k, be, nv: (be[_clamp(b, nv)], k, 0)),
                pl.BlockSpec((1, 1, D),
                             lambda b, k, be, nv: (be[_clamp(b, nv)], 0, 0)),
            ],
            out_specs=pl.BlockSpec((BLK, D), lambda b, k, be, nv: (b, 0)),
        ),
        out_shape=jax.ShapeDtypeStruct((RPAD, D), jnp.float32),
    )(*scalars, h2, W3, b3.reshape(NE, 1, D))

    return y


def kernel(x, Wg, bg, W1, b1, W2, b2, W3, b3):
    x2d = x.reshape(S, D)
    gate_logits = jnp.einsum('bsd,de->bse', x, Wg) + bg
    gate_weights = jax.nn.softmax(gate_logits, axis=-1)
    tv3, ti3 = jax.lax.top_k(gate_weights, TOPK)
    tv = tv3.reshape(S, TOPK)
    ti = ti3.reshape(S, TOPK)

    ti_flat = ti.reshape(-1)
    order = jnp.argsort(ti_flat, stable=True)
    e_sorted = ti_flat[order]
    counts = jnp.sum(
        (ti_flat[:, None] == jnp.arange(NE, dtype=jnp.int32)[None, :]).astype(
            jnp.int32), axis=0)
    nblk = (counts + BLK - 1) // BLK
    starts = jnp.cumsum(counts) - counts
    startsp = (jnp.cumsum(nblk) - nblk) * BLK
    r = jnp.arange(2 * S, dtype=jnp.int32)
    pos = startsp[e_sorted] + (r - starts[e_sorted])
    row_token = jnp.zeros((RPAD,), jnp.int32).at[pos].set(order // 2)
    inv = jnp.zeros((2 * S,), jnp.int32).at[order].set(pos).reshape(S, TOPK)
    cumblk = jnp.cumsum(nblk)
    nvalid = cumblk[-1:].astype(jnp.int32)
    block_expert = jnp.minimum(
        jnp.searchsorted(cumblk, jnp.arange(NBMAX), side='right'),
        NE - 1).astype(jnp.int32)

    xs = x2d[row_token]
    y = _grouped_mlp(xs, W1, b1, W2, b2, W3, b3, block_expert, nvalid)

    out = tv[:, 0:1] * y[inv[:, 0]] + tv[:, 1:2] * y[inv[:, 1]]
    return out.reshape(1, S, D)


# speedup vs baseline: 2.9042x; 1.0816x over previous
import functools

import jax
import jax.numpy as jnp
from jax import lax
from jax.experimental import pallas as pl
from jax.experimental.pallas import tpu as pltpu

S = 2048
D = 1024
H = 4096
NE = 8
TOPK = 2
BLK = 512
NBMAX = 16
RPAD = NBMAX * BLK
NT = 1024
KT = 1024


def _gelu(x):
    return 0.5 * x * (1.0 + lax.erf(x * 0.7071067811865476))


def _clamp(b, nv_ref):
    return jnp.minimum(b, nv_ref[0] - 1)


def _out_blk(b, nv_ref):
    return jnp.where(b < nv_ref[0], b, NBMAX)


def _out_n(b, n, nv_ref):
    return jnp.where(b < nv_ref[0], n, 0)


def _mlp1_body(be_ref, nv_ref, xs_ref, w1_ref, b1_ref, h1_ref):
    b = pl.program_id(0)

    @pl.when(b < nv_ref[0])
    def _():
        acc = jnp.dot(xs_ref[...], w1_ref[0].astype(jnp.bfloat16),
                      preferred_element_type=jnp.float32)
        h1_ref[...] = _gelu(acc + b1_ref[0]).astype(jnp.bfloat16)


def _mlp2_body(be_ref, nv_ref, h1_ref, w2_ref, b2_ref, h2_ref, acc_ref):
    b = pl.program_id(0)
    k = pl.program_id(2)
    nk = pl.num_programs(2)

    @pl.when(b < nv_ref[0])
    def _():
        off = pl.multiple_of(k * KT, KT)
        part = jnp.dot(h1_ref[:, pl.ds(off, KT)],
                       w2_ref[0].astype(jnp.bfloat16),
                       preferred_element_type=jnp.float32)

        @pl.when(k == 0)
        def _():
            acc_ref[...] = part

        @pl.when(k > 0)
        def _():
            acc_ref[...] += part

        @pl.when(k == nk - 1)
        def _():
            h2_ref[...] = _gelu(acc_ref[...] + b2_ref[0]).astype(jnp.bfloat16)


def _mlp3_body(be_ref, nv_ref, h2_ref, w3_ref, b3_ref, y_ref, acc_ref):
    b = pl.program_id(0)
    k = pl.program_id(1)
    nk = pl.num_programs(1)

    @pl.when(b < nv_ref[0])
    def _():
        off = pl.multiple_of(k * KT, KT)
        part = jnp.dot(h2_ref[:, pl.ds(off, KT)],
                       w3_ref[0].astype(jnp.bfloat16),
                       preferred_element_type=jnp.float32)

        @pl.when(k == 0)
        def _():
            acc_ref[...] = part

        @pl.when(k > 0)
        def _():
            acc_ref[...] += part

        @pl.when(k == nk - 1)
        def _():
            y_ref[...] = acc_ref[...] + b3_ref[0]


def _grouped_mlp(xs, W1, b1, W2, b2, W3, b3, block_expert, nvalid):
    scalars = (block_expert, nvalid)

    h1 = pl.pallas_call(
        _mlp1_body,
        grid_spec=pltpu.PrefetchScalarGridSpec(
            num_scalar_prefetch=2,
            grid=(NBMAX, H // NT),
            in_specs=[
                pl.BlockSpec((BLK, D), lambda b, n, be, nv: (_clamp(b, nv), 0)),
                pl.BlockSpec((1, D, NT),
                             lambda b, n, be, nv: (be[_clamp(b, nv)], 0, n)),
                pl.BlockSpec((1, 1, NT),
                             lambda b, n, be, nv: (be[_clamp(b, nv)], 0, n)),
            ],
            out_specs=pl.BlockSpec(
                (BLK, NT),
                lambda b, n, be, nv: (_out_blk(b, nv), _out_n(b, n, nv))),
        ),
        out_shape=jax.ShapeDtypeStruct((RPAD + BLK, H), jnp.bfloat16),
    )(*scalars, xs, W1, b1.reshape(NE, 1, H))

    h2 = pl.pallas_call(
        _mlp2_body,
        grid_spec=pltpu.PrefetchScalarGridSpec(
            num_scalar_prefetch=2,
            grid=(NBMAX, H // NT, H // KT),
            in_specs=[
                pl.BlockSpec((BLK, H),
                             lambda b, n, k, be, nv: (_clamp(b, nv), 0)),
                pl.BlockSpec((1, KT, NT),
                             lambda b, n, k, be, nv: (be[_clamp(b, nv)], k, n)),
                pl.BlockSpec((1, 1, NT),
                             lambda b, n, k, be, nv: (be[_clamp(b, nv)], 0, n)),
            ],
            out_specs=pl.BlockSpec(
                (BLK, NT),
                lambda b, n, k, be, nv: (_out_blk(b, nv), _out_n(b, n, nv))),
            scratch_shapes=[pltpu.VMEM((BLK, NT), jnp.float32)],
        ),
        out_shape=jax.ShapeDtypeStruct((RPAD + BLK, H), jnp.bfloat16),
    )(*scalars, h1, W2, b2.reshape(NE, 1, H))

    y = pl.pallas_call(
        _mlp3_body,
        grid_spec=pltpu.PrefetchScalarGridSpec(
            num_scalar_prefetch=2,
            grid=(NBMAX, H // KT),
            in_specs=[
                pl.BlockSpec((BLK, H), lambda b, k, be, nv: (_clamp(b, nv), 0)),
                pl.BlockSpec((1, KT, D),
                             lambda b, k, be, nv: (be[_clamp(b, nv)], k, 0)),
                pl.BlockSpec((1, 1, D),
                             lambda b, k, be, nv: (be[_clamp(b, nv)], 0, 0)),
            ],
            out_specs=pl.BlockSpec((BLK, D),
                                   lambda b, k, be, nv: (_out_blk(b, nv), 0)),
            scratch_shapes=[pltpu.VMEM((BLK, D), jnp.float32)],
        ),
        out_shape=jax.ShapeDtypeStruct((RPAD + BLK, D), jnp.float32),
    )(*scalars, h2, W3, b3.reshape(NE, 1, D))

    return y


def kernel(x, Wg, bg, W1, b1, W2, b2, W3, b3):
    x2d = x.reshape(S, D)
    gate_logits = jnp.einsum('bsd,de->bse', x, Wg) + bg
    gate_weights = jax.nn.softmax(gate_logits, axis=-1)
    tv3, ti3 = jax.lax.top_k(gate_weights, TOPK)
    tv = tv3.reshape(S, TOPK)
    ti = ti3.reshape(S, TOPK)

    ti_flat = ti.reshape(-1)
    order = jnp.argsort(ti_flat, stable=True)
    e_sorted = ti_flat[order]
    counts = jnp.sum(
        (ti_flat[:, None] == jnp.arange(NE, dtype=jnp.int32)[None, :]).astype(
            jnp.int32), axis=0)
    nblk = (counts + BLK - 1) // BLK
    starts = jnp.cumsum(counts) - counts
    startsp = (jnp.cumsum(nblk) - nblk) * BLK
    r = jnp.arange(2 * S, dtype=jnp.int32)
    pos = startsp[e_sorted] + (r - starts[e_sorted])
    row_token = jnp.zeros((RPAD,), jnp.int32).at[pos].set(order // 2)
    inv = jnp.zeros((2 * S,), jnp.int32).at[order].set(pos).reshape(S, TOPK)
    cumblk = jnp.cumsum(nblk)
    nvalid = cumblk[-1:].astype(jnp.int32)
    block_expert = jnp.minimum(
        jnp.searchsorted(cumblk, jnp.arange(NBMAX), side='right'),
        NE - 1).astype(jnp.int32)

    xs = x2d.astype(jnp.bfloat16)[row_token]
    y = _grouped_mlp(xs, W1, b1, W2, b2, W3, b3, block_expert, nvalid)

    out = tv[:, 0:1] * y[inv[:, 0]] + tv[:, 1:2] * y[inv[:, 1]]
    return out.reshape(1, S, D)


# speedup vs baseline: 3.7434x; 1.2890x over previous
import functools

import jax
import jax.numpy as jnp
from jax import lax
from jax.experimental import pallas as pl
from jax.experimental.pallas import tpu as pltpu

S = 2048
D = 1024
H = 4096
NE = 8
TOPK = 2
BLK = 512
NBMAX = 16
RPAD = NBMAX * BLK
NT = 1024


def _gelu(x):
    return 0.5 * x * (1.0 + lax.erf(x * 0.7071067811865476))


def _clamp(b, nv_ref):
    return jnp.minimum(b, nv_ref[0] - 1)


def _cast_weight_once(b, e, w_ref, wbf_ref, prev_ref):
    @pl.when((b == 0) | (e != prev_ref[0]))
    def _():
        wbf_ref[...] = w_ref[0].astype(jnp.bfloat16)
        prev_ref[0] = e


def _mlp1_body(be_ref, nv_ref, xs_ref, w1_ref, b1_ref, h1_ref,
               wbf_ref, prev_ref):
    b = pl.program_id(1)

    @pl.when(b < nv_ref[0])
    def _():
        e = be_ref[b]
        _cast_weight_once(b, e, w1_ref, wbf_ref, prev_ref)
        acc = jnp.dot(xs_ref[...], wbf_ref[...],
                      preferred_element_type=jnp.float32)
        h1_ref[...] = _gelu(acc + b1_ref[0]).astype(jnp.bfloat16)


def _mlp2_body(be_ref, nv_ref, h1_ref, w2_ref, b2_ref, h2_ref,
               wbf_ref, prev_ref):
    b = pl.program_id(1)

    @pl.when(b < nv_ref[0])
    def _():
        e = be_ref[b]
        _cast_weight_once(b, e, w2_ref, wbf_ref, prev_ref)
        acc = jnp.dot(h1_ref[...], wbf_ref[...],
                      preferred_element_type=jnp.float32)
        h2_ref[...] = _gelu(acc + b2_ref[0]).astype(jnp.bfloat16)


def _mlp3_body(be_ref, nv_ref, h2_ref, w3_ref, b3_ref, y_ref,
               wbf_ref, prev_ref):
    b = pl.program_id(0)

    @pl.when(b < nv_ref[0])
    def _():
        e = be_ref[b]
        _cast_weight_once(b, e, w3_ref, wbf_ref, prev_ref)
        acc = jnp.dot(h2_ref[...], wbf_ref[...],
                      preferred_element_type=jnp.float32)
        y_ref[...] = acc + b3_ref[0]


def _grouped_mlp(xs, W1, b1, W2, b2, W3, b3, block_expert, nvalid):
    scalars = (block_expert, nvalid)

    def _ospec(n, b, nv):
        return (jnp.where(b < nv[0], b, NBMAX), n)

    h1 = pl.pallas_call(
        _mlp1_body,
        grid_spec=pltpu.PrefetchScalarGridSpec(
            num_scalar_prefetch=2,
            grid=(H // NT, NBMAX),
            in_specs=[
                pl.BlockSpec((BLK, D), lambda n, b, be, nv: (_clamp(b, nv), 0)),
                pl.BlockSpec((1, D, NT),
                             lambda n, b, be, nv: (be[_clamp(b, nv)], 0, n)),
                pl.BlockSpec((1, 1, NT),
                             lambda n, b, be, nv: (be[_clamp(b, nv)], 0, n)),
            ],
            out_specs=pl.BlockSpec((BLK, NT),
                                   lambda n, b, be, nv: _ospec(n, b, nv)),
            scratch_shapes=[pltpu.VMEM((D, NT), jnp.bfloat16),
                            pltpu.SMEM((1,), jnp.int32)],
        ),
        out_shape=jax.ShapeDtypeStruct((RPAD + BLK, H), jnp.bfloat16),
    )(*scalars, xs, W1, b1.reshape(NE, 1, H))

    h2 = pl.pallas_call(
        _mlp2_body,
        grid_spec=pltpu.PrefetchScalarGridSpec(
            num_scalar_prefetch=2,
            grid=(H // NT, NBMAX),
            in_specs=[
                pl.BlockSpec((BLK, H), lambda n, b, be, nv: (_clamp(b, nv), 0)),
                pl.BlockSpec((1, H, NT),
                             lambda n, b, be, nv: (be[_clamp(b, nv)], 0, n)),
                pl.BlockSpec((1, 1, NT),
                             lambda n, b, be, nv: (be[_clamp(b, nv)], 0, n)),
            ],
            out_specs=pl.BlockSpec((BLK, NT),
                                   lambda n, b, be, nv: _ospec(n, b, nv)),
            scratch_shapes=[pltpu.VMEM((H, NT), jnp.bfloat16),
                            pltpu.SMEM((1,), jnp.int32)],
        ),
        out_shape=jax.ShapeDtypeStruct((RPAD + BLK, H), jnp.bfloat16),
    )(*scalars, h1, W2, b2.reshape(NE, 1, H))

    y = pl.pallas_call(
        _mlp3_body,
        grid_spec=pltpu.PrefetchScalarGridSpec(
            num_scalar_prefetch=2,
            grid=(NBMAX,),
            in_specs=[
                pl.BlockSpec((BLK, H), lambda b, be, nv: (_clamp(b, nv), 0)),
                pl.BlockSpec((1, H, D),
                             lambda b, be, nv: (be[_clamp(b, nv)], 0, 0)),
                pl.BlockSpec((1, 1, D),
                             lambda b, be, nv: (be[_clamp(b, nv)], 0, 0)),
            ],
            out_specs=pl.BlockSpec(
                (BLK, D),
                lambda b, be, nv: (jnp.where(b < nv[0], b, NBMAX), 0)),
            scratch_shapes=[pltpu.VMEM((H, D), jnp.bfloat16),
                            pltpu.SMEM((1,), jnp.int32)],
        ),
        out_shape=jax.ShapeDtypeStruct((RPAD + BLK, D), jnp.float32),
    )(*scalars, h2, W3, b3.reshape(NE, 1, D))

    return y


def kernel(x, Wg, bg, W1, b1, W2, b2, W3, b3):
    x2d = x.reshape(S, D)
    gate_logits = jnp.einsum('bsd,de->bse', x, Wg) + bg
    gate_weights = jax.nn.softmax(gate_logits, axis=-1)
    tv3, ti3 = jax.lax.top_k(gate_weights, TOPK)
    tv = tv3.reshape(S, TOPK)
    ti = ti3.reshape(S, TOPK)

    ti_flat = ti.reshape(-1)
    order = jnp.argsort(ti_flat, stable=True)
    e_sorted = ti_flat[order]
    counts = jnp.sum(
        (ti_flat[:, None] == jnp.arange(NE, dtype=jnp.int32)[None, :]).astype(
            jnp.int32), axis=0)
    nblk = (counts + BLK - 1) // BLK
    starts = jnp.cumsum(counts) - counts
    startsp = (jnp.cumsum(nblk) - nblk) * BLK
    r = jnp.arange(2 * S, dtype=jnp.int32)
    pos = startsp[e_sorted] + (r - starts[e_sorted])
    row_token = jnp.zeros((RPAD,), jnp.int32).at[pos].set(order // 2)
    inv = jnp.zeros((2 * S,), jnp.int32).at[order].set(pos).reshape(S, TOPK)
    cumblk = jnp.cumsum(nblk)
    nvalid = cumblk[-1:].astype(jnp.int32)
    block_expert = jnp.minimum(
        jnp.searchsorted(cumblk, jnp.arange(NBMAX), side='right'),
        NE - 1).astype(jnp.int32)

    xs = x2d.astype(jnp.bfloat16)[row_token]
    y = _grouped_mlp(xs, W1, b1, W2, b2, W3, b3, block_expert, nvalid)

    out = tv[:, 0:1] * y[inv[:, 0]] + tv[:, 1:2] * y[inv[:, 1]]
    return out.reshape(1, S, D)
